# baseline (speedup 1.0000x reference)
import jax
import jax.numpy as jnp
from jax.experimental import pallas as pl
from jax.experimental.pallas import tpu as pltpu

EMBED_DIM = 32
BLOCK_COLS = 128


def _mf_body(idx_ref, ublock_ref, iblock_ref, out_ref):
    u = idx_ref[0] % BLOCK_COLS
    i = idx_ref[1] % BLOCK_COLS
    lanes = jax.lax.broadcasted_iota(jnp.int32, (EMBED_DIM, BLOCK_COLS), 1)
    ucol = jnp.sum(jnp.where(lanes == u, ublock_ref[...], 0.0),
                   axis=1, keepdims=True)
    icol = jnp.sum(jnp.where(lanes == i, iblock_ref[...], 0.0),
                   axis=1, keepdims=True)
    out_ref[...] = jnp.sum(ucol * icol, axis=0, keepdims=True)


def kernel(user, item, users_emb, items_emb):
    idx = jnp.concatenate([user[None], item[None]])
    out = pl.pallas_call(
        _mf_body,
        grid_spec=pltpu.PrefetchScalarGridSpec(
            num_scalar_prefetch=1,
            grid=(1,),
            in_specs=[
                pl.BlockSpec((EMBED_DIM, BLOCK_COLS),
                             lambda g, idx: (0, idx[0] // BLOCK_COLS)),
                pl.BlockSpec((EMBED_DIM, BLOCK_COLS),
                             lambda g, idx: (0, idx[1] // BLOCK_COLS)),
            ],
            out_specs=pl.BlockSpec((1, 1), lambda g, idx: (0, 0)),
        ),
        out_shape=jax.ShapeDtypeStruct((1, 1), jnp.float32),
    )(idx, users_emb.T, items_emb.T)
    return out[0, 0]
